# Initial kernel scaffold; baseline (speedup 1.0000x reference)
import jax
import jax.numpy as jnp
from jax.experimental import pallas as pl


def kernel(x1, edge_index1, x2, edge_index2, x_litter, x_region, ei_ll, ei_rl, Wl1, Wr1, att1, b1, Wl2, Wr2, att2, b2, Wp_l, bp_l, Wp_r, bp_r, as_ll, ad_ll, as_rl, ad_rl, Wk, bk, q, Wh, bh, Wf, bf):
    raise NotImplementedError("write your pallas kernel here")


# speedup vs baseline: 15.1123x; 15.1123x over previous
import functools

import jax
import jax.numpy as jnp
from jax import lax
from jax.experimental import pallas as pl

_F32 = jnp.float32


def _full_max(a):
    return jnp.max(jnp.max(a, axis=1, keepdims=True), axis=0, keepdims=True)


def _onehot(iota, idx):
    return jnp.equal(iota, idx).astype(_F32)


def _dot(a, b):
    return jnp.dot(a, b, preferred_element_type=_F32)


def _gat_block(x, src_c, dst_c, dst_r, Wl, Wr, attf, S, R, bias, N, E):
    xl = _dot(x, Wl)
    xr = _dot(x, Wr)
    iota_en = lax.broadcasted_iota(jnp.int32, (E, N), 1)
    Msrc = _onehot(iota_en, src_c)
    G = _dot(Msrc, xl)
    Mdst = _onehot(iota_en, dst_c)
    m = G + _dot(Mdst, xr)
    t = jnp.where(m >= 0.0, m, 0.2 * m) * attf
    e = _dot(t, S)
    ex = jnp.exp(e - _full_max(e))
    iota_ne = lax.broadcasted_iota(jnp.int32, (N, E), 0)
    D = _onehot(iota_ne, dst_r)
    den = _dot(D, ex)
    alpha = ex / (_dot(Mdst, den) + 1e-16)
    out = _dot(D, _dot(alpha, R) * G)
    return out + bias


def _han_block(xs, xd, src_c, dst_c, dst_r, As, Ad, R8, Ns, Nd, E):
    a_s = _dot(xs, As)
    a_d = _dot(xd, Ad)
    iota_es = lax.broadcasted_iota(jnp.int32, (E, Ns), 1)
    Msrc = _onehot(iota_es, src_c)
    iota_ed = lax.broadcasted_iota(jnp.int32, (E, Nd), 1)
    Mdst = _onehot(iota_ed, dst_c)
    m = _dot(Msrc, a_s) + _dot(Mdst, a_d)
    e = jnp.where(m >= 0.0, m, 0.2 * m)
    ex = jnp.exp(e - _full_max(e))
    iota_ne = lax.broadcasted_iota(jnp.int32, (Nd, E), 0)
    D = _onehot(iota_ne, dst_r)
    den = _dot(D, ex)
    alpha = ex / (_dot(Mdst, den) + 1e-16)
    G = _dot(Msrc, xs)
    out = _dot(D, _dot(alpha, R8) * G)
    return jnp.maximum(out, 0.0)


def _fused_body(
    x1, x2, xlit, xreg,
    s1c, d1c, d1r, s2c, d2c, d2r,
    sllc, dllc, dllr, srlc, drlc, drlr,
    Wl1, Wr1, att1f, S1, R1, b1r,
    Wl2, Wr2, att2f, S2, R2, b2r,
    Wp_l, bp_lr, Wp_r, bp_rr,
    As_ll, Ad_ll, As_rl, Ad_rl, R8,
    Wk, bkr, q_row, Wh, bhr,
    y1, y2, y3,
):
    o1 = _gat_block(x1[...], s1c[...], d1c[...], d1r[...],
                    Wl1[...], Wr1[...], att1f[...], S1[...], R1[...],
                    b1r[...], 85, 765)
    y1[...] = jnp.maximum(o1, 0.0)

    o2 = _gat_block(x2[...], s2c[...], d2c[...], d2r[...],
                    Wl2[...], Wr2[...], att2f[...], S2[...], R2[...],
                    b2r[...], 438, 3942)
    y2[...] = jnp.maximum(o2, 0.0)

    h_l = _dot(xlit[...], Wp_l[...]) + bp_lr[...]
    h_r = _dot(xreg[...], Wp_r[...]) + bp_rr[...]
    o_ll = _han_block(h_l, h_l, sllc[...], dllc[...], dllr[...],
                      As_ll[...], Ad_ll[...], R8[...], 85, 85, 680)
    o_rl = _han_block(h_r, h_l, srlc[...], drlc[...], drlr[...],
                      As_rl[...], Ad_rl[...], R8[...], 60, 85, 1000)

    k0 = jnp.tanh(_dot(o_ll, Wk[...]) + bkr[...])
    k1 = jnp.tanh(_dot(o_rl, Wk[...]) + bkr[...])
    mean0 = jnp.sum(k0, axis=0, keepdims=True) * (1.0 / 85.0)
    mean1 = jnp.sum(k1, axis=0, keepdims=True) * (1.0 / 85.0)
    sc0 = jnp.sum(q_row[...] * mean0, axis=1, keepdims=True)
    sc1 = jnp.sum(q_row[...] * mean1, axis=1, keepdims=True)
    mx = jnp.maximum(sc0, sc1)
    e0 = jnp.exp(sc0 - mx)
    e1 = jnp.exp(sc1 - mx)
    inv = 1.0 / (e0 + e1)
    h = (e0 * inv) * o_ll + (e1 * inv) * o_rl
    y3[...] = jnp.maximum(_dot(h, Wh[...]) + bhr[...], 0.0)


def _final_body(x, Wf, bfr, o):
    o[...] = _dot(x[...], Wf[...]) + bfr[...]


@functools.partial(jax.jit, static_argnames=())
def kernel(x1, edge_index1, x2, edge_index2, x_litter, x_region, ei_ll, ei_rl,
           Wl1, Wr1, att1, b1, Wl2, Wr2, att2, b2,
           Wp_l, bp_l, Wp_r, bp_r, as_ll, ad_ll, as_rl, ad_rl,
           Wk, bk, q, Wh, bh, Wf, bf):
    f32 = _F32

    loops1 = jnp.arange(85, dtype=jnp.int32)
    src1 = jnp.concatenate([edge_index1[0], loops1])
    dst1 = jnp.concatenate([edge_index1[1], loops1])
    loops2 = jnp.arange(438, dtype=jnp.int32)
    src2 = jnp.concatenate([edge_index2[0], loops2])
    dst2 = jnp.concatenate([edge_index2[1], loops2])

    def col(v):
        return v.reshape(-1, 1)

    def row(v):
        return v.reshape(1, -1)

    R1 = jnp.kron(jnp.eye(2, dtype=f32), jnp.ones((1, 60), dtype=f32))
    S1 = R1.T
    R2 = jnp.kron(jnp.eye(2, dtype=f32), jnp.ones((1, 10), dtype=f32))
    S2 = R2.T
    R8 = jnp.kron(jnp.eye(8, dtype=f32), jnp.ones((1, 16), dtype=f32))
    K8 = R8.T
    As_ll = as_ll.reshape(128, 1) * K8
    Ad_ll = ad_ll.reshape(128, 1) * K8
    As_rl = as_rl.reshape(128, 1) * K8
    Ad_rl = ad_rl.reshape(128, 1) * K8

    args = (
        x1, x2, x_litter, x_region,
        col(src1), col(dst1), row(dst1),
        col(src2), col(dst2), row(dst2),
        col(ei_ll[0]), col(ei_ll[1]), row(ei_ll[1]),
        col(ei_rl[0]), col(ei_rl[1]), row(ei_rl[1]),
        Wl1, Wr1, att1.reshape(1, 120), S1, R1, b1.reshape(1, 120),
        Wl2, Wr2, att2.reshape(1, 20), S2, R2, b2.reshape(1, 20),
        Wp_l, bp_l.reshape(1, 128), Wp_r, bp_r.reshape(1, 128),
        As_ll, Ad_ll, As_rl, Ad_rl, R8,
        Wk, bk.reshape(1, 128), q.reshape(1, 128), Wh, bh.reshape(1, 120),
    )

    y1, y2, y3 = pl.pallas_call(
        _fused_body,
        out_shape=(
            jax.ShapeDtypeStruct((85, 120), f32),
            jax.ShapeDtypeStruct((438, 20), f32),
            jax.ShapeDtypeStruct((85, 120), f32),
        ),
    )(*args)

    xcat = jnp.concatenate(
        [y1.reshape(120, 85), y2.reshape(120, 73), y3.reshape(120, 85)], axis=1)

    out = pl.pallas_call(
        _final_body,
        out_shape=jax.ShapeDtypeStruct((120, 5), f32),
    )(xcat, Wf, bf.reshape(1, 5))
    return out


# speedup vs baseline: 19.4937x; 1.2899x over previous
import jax
import jax.numpy as jnp
from jax import lax
from jax.experimental import pallas as pl

_F32 = jnp.float32
_BF16 = jnp.bfloat16
_DNT = (((0,), (0,)), ((), ()))


def _full_max(a):
    return jnp.max(jnp.max(a, axis=1, keepdims=True), axis=0, keepdims=True)


def _onehot(iota, idx):
    return jnp.equal(iota, idx).astype(_BF16)


def _dot(a, b):
    return jnp.dot(a, b, preferred_element_type=_F32)


def _split(v):
    hi = v.astype(_BF16)
    return hi, (v - hi.astype(_F32)).astype(_BF16)


def _bdot(o, v):
    vh, vl = _split(v)
    return _dot(o, vh) + _dot(o, vl)


def _bdot_t(o, v):
    vh, vl = _split(v)
    return (lax.dot_general(o, vh, _DNT, preferred_element_type=_F32)
            + lax.dot_general(o, vl, _DNT, preferred_element_type=_F32))


def _lrelu(m):
    return jnp.where(m >= 0.0, m, 0.2 * m)


def _gat_block(x, ei, Wl, Wr, attf, S, R, bias, N, E):
    xl = _dot(x, Wl)
    xr = _dot(x, Wr)
    src_c = jnp.swapaxes(ei[0:1, :], 0, 1)
    dst_c = jnp.swapaxes(ei[1:2, :], 0, 1)
    iota_en = lax.broadcasted_iota(jnp.int32, (E, N), 1)
    Msrc = _onehot(iota_en, src_c)
    Mdst = _onehot(iota_en, dst_c)
    iota_ne = lax.broadcasted_iota(jnp.int32, (N, E), 0)
    D = _onehot(iota_ne, ei[1:2, :])
    G = _bdot(Msrc, xl)
    m = G + _bdot(Mdst, xr)
    t = _lrelu(m) * attf
    t_loop = _lrelu(xl + xr) * attf
    e = _bdot(t, S)
    e_loop = _bdot(t_loop, S)
    C = jnp.maximum(_full_max(e), _full_max(e_loop))
    ex = jnp.exp(e - C)
    ex_loop = jnp.exp(e_loop - C)
    den = _bdot(D, ex) + ex_loop
    alpha = ex / (_bdot(Mdst, den) + 1e-16)
    alpha_loop = ex_loop / (den + 1e-16)
    w = _bdot(alpha, R) * G
    w_loop = _bdot(alpha_loop, R) * xl
    out = _bdot(D, w) + w_loop
    return out + bias


def _han_block(xs, xd, ei, att_s, att_d, K8, R8, Ns, Nd, E):
    a_s = _bdot(xs * att_s, K8)
    a_d = _bdot(xd * att_d, K8)
    src_c = jnp.swapaxes(ei[0:1, :], 0, 1)
    dst_c = jnp.swapaxes(ei[1:2, :], 0, 1)
    iota_es = lax.broadcasted_iota(jnp.int32, (E, Ns), 1)
    Msrc = _onehot(iota_es, src_c)
    iota_ed = lax.broadcasted_iota(jnp.int32, (E, Nd), 1)
    Mdst = _onehot(iota_ed, dst_c)
    iota_ne = lax.broadcasted_iota(jnp.int32, (Nd, E), 0)
    D = _onehot(iota_ne, ei[1:2, :])
    e = _lrelu(_bdot(Msrc, a_s) + _bdot(Mdst, a_d))
    ex = jnp.exp(e - _full_max(e))
    den = _bdot(D, ex)
    alpha = ex / (_bdot(Mdst, den) + 1e-16)
    G = _bdot(Msrc, xs)
    out = _bdot(D, _bdot(alpha, R8) * G)
    return jnp.maximum(out, 0.0)


def _flatten_rows(a):
    H = a.shape[0]
    return jnp.concatenate([a[i:i + 1, :] for i in range(H)], axis=1)


def _fused_body(
    x1r, ei1, x2r, ei2, xlit, xreg, eill, eirl,
    Wl1, Wr1, att1, b1v, Wl2, Wr2, att2, b2v,
    S1, R1, S2, R2, K8, R8,
    Wp_l, bp_lv, Wp_r, bp_rv,
    as_llr, ad_llr, as_rlr, ad_rlr,
    Wk, bkv, qv, Wh, bhv,
    y1o, y2o, y3o,
):
    att1f = _flatten_rows(att1[...])
    att2f = _flatten_rows(att2[...])
    o1 = _gat_block(x1r[...], ei1[...], Wl1[...], Wr1[...], att1f,
                    S1[...], R1[...], b1v[...], 85, 680)
    y1o[...] = jnp.maximum(o1, 0.0)
    o2 = _gat_block(x2r[...], ei2[...], Wl2[...], Wr2[...], att2f,
                    S2[...], R2[...], b2v[...], 438, 3504)
    y2o[...] = jnp.maximum(o2, 0.0)

    h_l = _dot(xlit[...], Wp_l[...]) + bp_lv[...]
    h_r = _dot(xreg[...], Wp_r[...]) + bp_rv[...]
    o_ll = _han_block(h_l, h_l, eill[...], _flatten_rows(as_llr[...]),
                      _flatten_rows(ad_llr[...]), K8[...], R8[...], 85, 85, 680)
    o_rl = _han_block(h_r, h_l, eirl[...], _flatten_rows(as_rlr[...]),
                      _flatten_rows(ad_rlr[...]), K8[...], R8[...], 60, 85, 1000)

    k0 = jnp.tanh(_dot(o_ll, Wk[...]) + bkv[...])
    k1 = jnp.tanh(_dot(o_rl, Wk[...]) + bkv[...])
    mean0 = jnp.sum(k0, axis=0, keepdims=True) * (1.0 / 85.0)
    mean1 = jnp.sum(k1, axis=0, keepdims=True) * (1.0 / 85.0)
    sc0 = jnp.sum(qv[...] * mean0, axis=1, keepdims=True)
    sc1 = jnp.sum(qv[...] * mean1, axis=1, keepdims=True)
    mx = jnp.maximum(sc0, sc1)
    e0 = jnp.exp(sc0 - mx)
    e1 = jnp.exp(sc1 - mx)
    inv = 1.0 / (e0 + e1)
    h = (e0 * inv) * o_ll + (e1 * inv) * o_rl
    y3o[...] = jnp.maximum(_dot(h, Wh[...]) + bhv[...], 0.0)


def _final_body(x, Wf, bfv, o):
    o[...] = _dot(x[...], Wf[...]) + bfv[...]


def kernel(x1, edge_index1, x2, edge_index2, x_litter, x_region, ei_ll, ei_rl,
           Wl1, Wr1, att1, b1, Wl2, Wr2, att2, b2,
           Wp_l, bp_l, Wp_r, bp_r, as_ll, ad_ll, as_rl, ad_rl,
           Wk, bk, q, Wh, bh, Wf, bf):
    f32 = _F32

    R1 = jnp.kron(jnp.eye(2, dtype=f32), jnp.ones((1, 60), dtype=f32)).astype(_BF16)
    S1 = jnp.kron(jnp.eye(2, dtype=f32), jnp.ones((60, 1), dtype=f32)).astype(_BF16)
    R2 = jnp.kron(jnp.eye(2, dtype=f32), jnp.ones((1, 10), dtype=f32)).astype(_BF16)
    S2 = jnp.kron(jnp.eye(2, dtype=f32), jnp.ones((10, 1), dtype=f32)).astype(_BF16)
    R8 = jnp.kron(jnp.eye(8, dtype=f32), jnp.ones((1, 16), dtype=f32)).astype(_BF16)
    K8 = jnp.kron(jnp.eye(8, dtype=f32), jnp.ones((16, 1), dtype=f32)).astype(_BF16)

    y1, y2, y3 = pl.pallas_call(
        _fused_body,
        out_shape=(
            jax.ShapeDtypeStruct((85, 120), f32),
            jax.ShapeDtypeStruct((438, 20), f32),
            jax.ShapeDtypeStruct((85, 120), f32),
        ),
    )(x1, edge_index1, x2, edge_index2, x_litter, x_region, ei_ll, ei_rl,
      Wl1, Wr1, att1, b1, Wl2, Wr2, att2, b2,
      S1, R1, S2, R2, K8, R8,
      Wp_l, bp_l, Wp_r, bp_r, as_ll, ad_ll, as_rl, ad_rl,
      Wk, bk, q, Wh, bh)

    xcat = jnp.concatenate(
        [y1.reshape(120, 85), y2.reshape(120, 73), y3.reshape(120, 85)], axis=1)

    return pl.pallas_call(
        _final_body,
        out_shape=jax.ShapeDtypeStruct((120, 5), f32),
    )(xcat, Wf, bf)


# speedup vs baseline: 30.3103x; 1.5549x over previous
import jax
import jax.numpy as jnp
from jax import lax
from jax.experimental import pallas as pl

_F32 = jnp.float32


def _full_max(a):
    return jnp.max(jnp.max(a, axis=1, keepdims=True), axis=0, keepdims=True)


def _onehot(iota, idx):
    return jnp.equal(iota, idx).astype(_F32)


def _dot(a, b):
    return jnp.dot(a, b, preferred_element_type=_F32)


def _lrelu(m):
    return jnp.where(m >= 0.0, m, 0.2 * m)


def _t(a):
    return jnp.swapaxes(a, 0, 1)


def _flatten_rows(a):
    H = a.shape[0]
    return jnp.concatenate([a[i:i + 1, :] for i in range(H)], axis=1)


def _gat_block(xT, ei, Wl, Wr, att, SUM, EXP, bias, N, E):
    xlT = _dot(_t(Wl), xT)
    xrT = _dot(_t(Wr), xT)
    attc = _t(_flatten_rows(att))
    iota_ne = lax.broadcasted_iota(jnp.int32, (N, E), 0)
    MsrcT = _onehot(iota_ne, ei[0:1, :])
    MdstT = _onehot(iota_ne, ei[1:2, :])
    iota_en = lax.broadcasted_iota(jnp.int32, (E, N), 1)
    Mdst = _onehot(iota_en, _t(ei[1:2, :]))
    GT = _dot(xlT, MsrcT)
    mT = GT + _dot(xrT, MdstT)
    tT = _lrelu(mT) * attc
    tT_loop = _lrelu(xlT + xrT) * attc
    eT = _dot(SUM, tT)
    eT_loop = _dot(SUM, tT_loop)
    C = jnp.maximum(_full_max(eT), _full_max(eT_loop))
    exT = jnp.exp(eT - C)
    exT_loop = jnp.exp(eT_loop - C)
    denT = _dot(exT, Mdst) + exT_loop
    alphaT = exT / (_dot(denT, MdstT) + 1e-16)
    alphaT_loop = exT_loop / (denT + 1e-16)
    wT = _dot(EXP, alphaT) * GT
    wT_loop = _dot(EXP, alphaT_loop) * xlT
    outT = _dot(wT, Mdst) + wT_loop
    return _t(outT) + bias


def _han_block(xsT, xdT, ei, att_s, att_d, K8, R8, Ns, Nd, E):
    a_sT = _dot(R8, xsT * _t(_flatten_rows(att_s)))
    a_dT = _dot(R8, xdT * _t(_flatten_rows(att_d)))
    iota_se = lax.broadcasted_iota(jnp.int32, (Ns, E), 0)
    MsrcT = _onehot(iota_se, ei[0:1, :])
    iota_de = lax.broadcasted_iota(jnp.int32, (Nd, E), 0)
    MdstT = _onehot(iota_de, ei[1:2, :])
    iota_ed = lax.broadcasted_iota(jnp.int32, (E, Nd), 1)
    Mdst = _onehot(iota_ed, _t(ei[1:2, :]))
    eT = _lrelu(_dot(a_sT, MsrcT) + _dot(a_dT, MdstT))
    exT = jnp.exp(eT - _full_max(eT))
    denT = _dot(exT, Mdst)
    alphaT = exT / (_dot(denT, MdstT) + 1e-16)
    GT = _dot(xsT, MsrcT)
    outT = _dot(_dot(K8, alphaT) * GT, Mdst)
    return jnp.maximum(outT, 0.0)


def _fused_body(
    x1r, ei1, x2r, ei2, xlit, xreg, eill, eirl,
    Wl1, Wr1, att1, b1v, Wl2, Wr2, att2, b2v,
    SUM1, EXP1, SUM2, EXP2, K8, R8,
    Wp_l, bp_lv, Wp_r, bp_rv,
    as_llr, ad_llr, as_rlr, ad_rlr,
    Wk, bkv, qv, Wh, bhv,
    y1o, y2o, y3o,
):
    o1 = _gat_block(_t(x1r[...]), ei1[...], Wl1[...], Wr1[...], att1[...],
                    SUM1[...], EXP1[...], b1v[...], 85, 680)
    y1o[...] = jnp.maximum(o1, 0.0)
    o2 = _gat_block(_t(x2r[...]), ei2[...], Wl2[...], Wr2[...], att2[...],
                    SUM2[...], EXP2[...], b2v[...], 438, 3504)
    y2o[...] = jnp.maximum(o2, 0.0)

    bplc = _t(bp_lv[...].reshape(1, 128))
    bprc = _t(bp_rv[...].reshape(1, 128))
    h_lT = _dot(_t(Wp_l[...]), _t(xlit[...])) + bplc
    h_rT = _dot(_t(Wp_r[...]), _t(xreg[...])) + bprc
    o_llT = _han_block(h_lT, h_lT, eill[...], as_llr[...], ad_llr[...],
                       K8[...], R8[...], 85, 85, 680)
    o_rlT = _han_block(h_rT, h_lT, eirl[...], as_rlr[...], ad_rlr[...],
                       K8[...], R8[...], 60, 85, 1000)

    bkc = _t(bkv[...].reshape(1, 128))
    qc = _t(qv[...].reshape(1, 128))
    k0 = jnp.tanh(_dot(_t(Wk[...]), o_llT) + bkc)
    k1 = jnp.tanh(_dot(_t(Wk[...]), o_rlT) + bkc)
    mean0 = jnp.sum(k0, axis=1, keepdims=True) * (1.0 / 85.0)
    mean1 = jnp.sum(k1, axis=1, keepdims=True) * (1.0 / 85.0)
    sc0 = jnp.sum(qc * mean0, axis=0, keepdims=True)
    sc1 = jnp.sum(qc * mean1, axis=0, keepdims=True)
    mx = jnp.maximum(sc0, sc1)
    e0 = jnp.exp(sc0 - mx)
    e1 = jnp.exp(sc1 - mx)
    inv = 1.0 / (e0 + e1)
    hT = (e0 * inv) * o_llT + (e1 * inv) * o_rlT
    y3T = _dot(_t(Wh[...]), hT)
    y3o[...] = jnp.maximum(_t(y3T) + bhv[...], 0.0)


def _final_body(x, Wf, bfv, o):
    o[...] = _dot(x[...], Wf[...]) + bfv[...]


def kernel(x1, edge_index1, x2, edge_index2, x_litter, x_region, ei_ll, ei_rl,
           Wl1, Wr1, att1, b1, Wl2, Wr2, att2, b2,
           Wp_l, bp_l, Wp_r, bp_r, as_ll, ad_ll, as_rl, ad_rl,
           Wk, bk, q, Wh, bh, Wf, bf):
    f32 = _F32

    SUM1 = jnp.kron(jnp.eye(2, dtype=f32), jnp.ones((1, 60), dtype=f32))
    EXP1 = jnp.kron(jnp.eye(2, dtype=f32), jnp.ones((60, 1), dtype=f32))
    SUM2 = jnp.kron(jnp.eye(2, dtype=f32), jnp.ones((1, 10), dtype=f32))
    EXP2 = jnp.kron(jnp.eye(2, dtype=f32), jnp.ones((10, 1), dtype=f32))
    R8 = jnp.kron(jnp.eye(8, dtype=f32), jnp.ones((1, 16), dtype=f32))
    K8 = jnp.kron(jnp.eye(8, dtype=f32), jnp.ones((16, 1), dtype=f32))

    y1, y2, y3 = pl.pallas_call(
        _fused_body,
        out_shape=(
            jax.ShapeDtypeStruct((85, 120), f32),
            jax.ShapeDtypeStruct((438, 20), f32),
            jax.ShapeDtypeStruct((85, 120), f32),
        ),
    )(x1, edge_index1, x2, edge_index2, x_litter, x_region, ei_ll, ei_rl,
      Wl1, Wr1, att1, b1, Wl2, Wr2, att2, b2,
      SUM1, EXP1, SUM2, EXP2, K8, R8,
      Wp_l, bp_l, Wp_r, bp_r, as_ll, ad_ll, as_rl, ad_rl,
      Wk, bk, q, Wh, bh)

    xcat = jnp.concatenate(
        [y1.reshape(120, 85), y2.reshape(120, 73), y3.reshape(120, 85)], axis=1)

    return pl.pallas_call(
        _final_body,
        out_shape=jax.ShapeDtypeStruct((120, 5), f32),
    )(xcat, Wf, bf)
